# baseline (speedup 1.0000x reference)
import jax
import jax.numpy as jnp
from jax import lax
from jax.experimental import pallas as pl
from jax.experimental.pallas import tpu as pltpu
from jax.experimental.pallas import tpu_sc as plsc

_INFO = plsc.get_sparse_core_info()
_NC = _INFO.num_cores
_NS = _INFO.num_subcores
_L = _INFO.num_lanes
_NW = _NC * _NS

_BATCH = 4096
_NNEG = 50
_D = 128
_CH = _D // _L
_BPW = _BATCH // _NW
_NG = _BPW // _L
_JC = 3
_NFULL = _NNEG // _JC
_JTAIL = _NNEG - _NFULL * _JC
_NCHUNK = _NFULL + 1
_NIDPAD = _NCHUNK * _JC


def _body(aid_hbm, pid_hbm, nidT_hbm, users_hbm, items_hbm, out_hbm,
          aidx_v, pidx_v, nidx_v, a_v, bufs, pd_v, outc, sems):
    wid = lax.axis_index("s") * _NC + lax.axis_index("c")
    base = wid * _BPW

    pltpu.sync_copy(aid_hbm.at[pl.ds(base, _BPW)], aidx_v)
    pltpu.sync_copy(pid_hbm.at[pl.ds(base, _BPW)], pidx_v)
    pltpu.sync_copy(nidT_hbm.at[:, pl.ds(base, _BPW)],
                    nidx_v.at[pl.ds(0, _NNEG)])
    pltpu.sync_copy(nidT_hbm.at[0, pl.ds(base, _BPW)], nidx_v.at[_NNEG])

    def fire(ch, b):
        for jl in range(_JC):
            pltpu.async_copy(
                items_hbm.at[nidx_v.at[ch * _JC + jl]],
                bufs[b].at[pl.ds(jl * _BPW, _BPW)], sems[b])

    def drain(b):
        for jl in range(_JC):
            pltpu.make_async_copy(
                items_hbm.at[nidx_v.at[0]],
                bufs[b].at[pl.ds(jl * _BPW, _BPW)], sems[b]).wait()

    cp_a = pltpu.async_copy(users_hbm.at[aidx_v], a_v, sems[2])
    cp_p = pltpu.async_copy(
        items_hbm.at[pidx_v], bufs[1].at[pl.ds(0, _BPW)], sems[1])
    fire(0, 0)

    i0 = lax.iota(jnp.int32, _L)
    perms = [i0 ^ 8, i0 ^ 4, i0 ^ 2, i0 ^ 1]

    def butterfly(acc):
        for p in perms:
            acc = acc + acc.at[p].get(mode="promise_in_bounds")
        return acc

    cp_a.wait()
    cp_p.wait()

    def pg(g, carry):
        res = jnp.zeros((_L,), jnp.float32)
        for rl in range(_L):
            r = g * _L + rl
            acc = jnp.zeros((_L,), jnp.float32)
            for c in range(_CH):
                a = a_v[r, pl.ds(c * _L, _L)]
                b = bufs[1][r, pl.ds(c * _L, _L)]
                d = a - b
                acc = acc + d * d
            res = jnp.where(i0 == rl, butterfly(acc), res)
        pd_v[pl.ds(g * _L, _L)] = res
        return carry

    lax.fori_loop(0, _NG, pg, 0)
    fire(1, 1)

    def chunk_compute(buf, ch, oc, njl):
        def ng(g, c2):
            res = [jnp.zeros((_L,), jnp.float32) for _ in range(njl)]
            for rl in range(_L):
                r = g * _L + rl
                a = [a_v[r, pl.ds(c * _L, _L)] for c in range(_CH)]
                for jl in range(njl):
                    acc = jnp.zeros((_L,), jnp.float32)
                    for c in range(_CH):
                        b = buf[jl * _BPW + r, pl.ds(c * _L, _L)]
                        d = a[c] - b
                        acc = acc + d * d
                    res[jl] = jnp.where(i0 == rl, butterfly(acc), res[jl])
            pd = pd_v[pl.ds(g * _L, _L)]
            for jl in range(njl):
                oc[jl, pl.ds(g * _L, _L)] = pd - res[jl]
            return c2

        lax.fori_loop(0, _NG, ng, 0)

    def put_out(ch, oc, sem, njl):
        for jl in range(njl):
            off = pl.multiple_of((ch * _JC + jl) * _BATCH + base, _BPW)
            pltpu.async_copy(oc.at[jl], out_hbm.at[pl.ds(off, _BPW)], sem)

    def drain_out(sem, n):
        for _ in range(n):
            pltpu.make_async_copy(
                outc[0].at[0], out_hbm.at[pl.ds(0, _BPW)], sem).wait()

    def duo(m, carry):
        ch0 = m * 2
        for b in range(2):
            ch = ch0 + b

            @pl.when(ch >= 2)
            def _():
                drain_out(sems[3 + b], _JC)

            drain(b)
            chunk_compute(bufs[b], ch, outc[b], _JC)

            @pl.when(ch + 2 < _NCHUNK)
            def _():
                fire(ch + 2, b)

            put_out(ch, outc[b], sems[3 + b], _JC)
        return carry

    lax.fori_loop(0, _NFULL // 2, duo, 0)

    drain_out(sems[3], _JC)
    drain(0)
    chunk_compute(bufs[0], _NFULL, outc[0], _JTAIL)
    put_out(_NFULL, outc[0], sems[3], _JTAIL)
    drain_out(sems[4], _JC)
    drain_out(sems[3], _JTAIL)


@jax.jit
def _pairwise_sc(anchor_ids, pos_ids, negT_ids, users, items):
    mesh = plsc.VectorSubcoreMesh(core_axis_name="c", subcore_axis_name="s")

    def body(aid, pid, nid, u, it, out, aidx, pidx, nidx, a_v,
             b0, b1, pd_v, oc0, oc1, s0, s1, s2, s3, s4):
        _body(aid, pid, nid, u, it, out, aidx, pidx, nidx, a_v,
              (b0, b1), pd_v, (oc0, oc1), (s0, s1, s2, s3, s4))

    fn = pl.kernel(
        body,
        mesh=mesh,
        out_type=jax.ShapeDtypeStruct((_NNEG * _BATCH,), jnp.float32),
        scratch_types=[
            pltpu.VMEM((_BPW,), jnp.int32),
            pltpu.VMEM((_BPW,), jnp.int32),
            pltpu.VMEM((_NIDPAD, _BPW), jnp.int32),
            pltpu.VMEM((_BPW, _D), jnp.float32),
            pltpu.VMEM((_JC * _BPW, _D), jnp.float32),
            pltpu.VMEM((_JC * _BPW, _D), jnp.float32),
            pltpu.VMEM((_BPW,), jnp.float32),
            pltpu.VMEM((_JC, _BPW), jnp.float32),
            pltpu.VMEM((_JC, _BPW), jnp.float32),
            pltpu.SemaphoreType.DMA,
            pltpu.SemaphoreType.DMA,
            pltpu.SemaphoreType.DMA,
            pltpu.SemaphoreType.DMA,
            pltpu.SemaphoreType.DMA,
        ],
    )
    return fn(anchor_ids, pos_ids, negT_ids, users, items)


def kernel(anchor_ids, pos_ids, neg_ids, users, items):
    negT = neg_ids.T
    outf = _pairwise_sc(anchor_ids, pos_ids, negT, users, items)
    return outf.reshape(_NNEG, _BATCH).T
